# Initial kernel scaffold; baseline (speedup 1.0000x reference)
import jax
import jax.numpy as jnp
from jax.experimental import pallas as pl


def kernel(ploc, plabel, gloc, glabel, dboxes):
    raise NotImplementedError("write your pallas kernel here")


# speedup vs baseline: 1.7795x; 1.7795x over previous
import functools

import jax
import jax.numpy as jnp
from jax.experimental import pallas as pl
from jax.experimental.pallas import tpu as pltpu

N, C, L = 64, 81, 8732
SCALE_XY = 10.0
SCALE_WH = 5.0
INT32_MIN = -2147483648
INT32_MAX = 2147483647


def _sortable_key(f):
    b = jax.lax.bitcast_convert_type(f, jnp.int32)
    return jnp.where(b >= 0, b, jnp.int32(INT32_MIN) - b)


def _loss_kernel(plabel_ref, glabel_ref, ploc_ref, gloc_ref, dboxes_ref, out_ref):
    i = pl.program_id(0)

    x = plabel_ref[0]
    labels = glabel_ref[0]

    m = jnp.max(x, axis=0, keepdims=True)
    s = jnp.sum(jnp.exp(x - m), axis=0, keepdims=True)
    lse = jnp.log(s) + m

    cls = jax.lax.broadcasted_iota(jnp.int32, (C, L), 0)
    onehot = (cls == labels).astype(jnp.float32)
    picked = jnp.sum(onehot * x, axis=0, keepdims=True)
    con = lse - picked

    mask = labels > 0
    maskf = mask.astype(jnp.float32)
    pos = jnp.sum(labels > 0).astype(jnp.int32)

    p = ploc_ref[0]
    g = gloc_ref[0]
    d = dboxes_ref[0]
    gxy = SCALE_XY * (g[:2, :] - d[:2, :]) / d[2:, :]
    gwh = SCALE_WH * jnp.log(g[2:, :] / d[2:, :])
    dxy = p[:2, :] - gxy
    dwh = p[2:, :] - gwh
    diff = jnp.concatenate([dxy, dwh], axis=0)
    ad = jnp.abs(diff)
    sl1 = jnp.sum(jnp.where(ad < 1.0, 0.5 * diff * diff, ad - 0.5), axis=0,
                  keepdims=True)
    sl1_pos = jnp.sum(maskf * sl1)

    con_neg = jnp.where(mask, 0.0, con)
    keys = _sortable_key(con_neg)
    K = jnp.minimum(3 * pos, jnp.int32(L))

    cnt_nonneg = jnp.sum((keys >= 0).astype(jnp.int32))
    lo0 = jnp.where(cnt_nonneg >= K, jnp.int32(0), jnp.int32(INT32_MIN))
    hi0 = jnp.where(cnt_nonneg >= K, jnp.int32(INT32_MAX), jnp.int32(-1))

    def val_body(_, carry):
        lo, hi = carry
        span = hi - lo
        mid = lo + (span >> 1) + (span & 1)
        cnt = jnp.sum((keys >= mid).astype(jnp.int32))
        ok = cnt >= K
        return jnp.where(ok, mid, lo), jnp.where(ok, hi, mid - 1)

    t, _ = jax.lax.fori_loop(0, 31, val_body, (lo0, hi0))

    gt = keys > t
    count_gt = jnp.sum(gt.astype(jnp.int32))
    need = K - count_gt

    tied = keys == t
    tied_i32 = tied.astype(jnp.int32)
    idx = jax.lax.broadcasted_iota(jnp.int32, (1, L), 1)

    def idx_body(_, carry):
        lo, hi = carry
        mid = (lo + hi) >> 1
        cnt = jnp.sum(jnp.where(idx < mid, tied_i32, 0))
        ok = cnt >= need
        return jnp.where(ok, lo, mid + 1), jnp.where(ok, mid, hi)

    msel, _ = jax.lax.fori_loop(0, 14, idx_body, (jnp.int32(0), jnp.int32(L)))

    neg_sel = jnp.logical_or(gt, jnp.logical_and(tied, idx < msel))
    closs = jnp.sum(con * (maskf + neg_sel.astype(jnp.float32)))

    total = sl1_pos + closs
    posf = pos.astype(jnp.float32)
    num_mask = (pos > 0).astype(jnp.float32)
    contrib = total * num_mask / jnp.maximum(posf, 1e-6) / N

    @pl.when(i == 0)
    def _():
        out_ref[...] = jnp.zeros((1, 1), jnp.float32)

    out_ref[...] += jnp.full((1, 1), contrib, jnp.float32)


@jax.jit
def kernel(ploc, plabel, gloc, glabel, dboxes):
    ploc = ploc.astype(jnp.float32)
    plabel = plabel.astype(jnp.float32)
    glabel3 = glabel.reshape(N, 1, L).astype(jnp.int32)

    out = pl.pallas_call(
        _loss_kernel,
        grid=(N,),
        in_specs=[
            pl.BlockSpec((1, C, L), lambda i: (i, 0, 0)),
            pl.BlockSpec((1, 1, L), lambda i: (i, 0, 0)),
            pl.BlockSpec((1, 4, L), lambda i: (i, 0, 0)),
            pl.BlockSpec((1, 4, L), lambda i: (i, 0, 0)),
            pl.BlockSpec((1, 4, L), lambda i: (0, 0, 0)),
        ],
        out_specs=pl.BlockSpec((1, 1), lambda i: (0, 0)),
        out_shape=jax.ShapeDtypeStruct((1, 1), jnp.float32),
    )(plabel, glabel3, ploc, gloc, dboxes)
    return out[0, 0]


# speedup vs baseline: 4.8146x; 2.7056x over previous
import jax
import jax.numpy as jnp
from jax.experimental import pallas as pl

N, C, L = 64, 81, 8732
SCALE_XY = 10.0
SCALE_WH = 5.0
INT32_MIN = -2147483648
INT32_MAX = 2147483647


def _sortable_key(f):
    b = jax.lax.bitcast_convert_type(f, jnp.int32)
    return jnp.where(b >= 0, b, jnp.int32(INT32_MIN) - b)


def _dense_kernel(plabel_ref, glabel_ref, ploc_ref, gloc_ref, dboxes_ref,
                  con_ref, sl1_ref):
    x = plabel_ref[0]
    labels = glabel_ref[0]

    m = jnp.max(x, axis=0, keepdims=True)
    s = jnp.sum(jnp.exp(x - m), axis=0, keepdims=True)
    lse = jnp.log(s) + m

    cls = jax.lax.broadcasted_iota(jnp.int32, (C, L), 0)
    onehot = (cls == labels).astype(jnp.float32)
    picked = jnp.sum(onehot * x, axis=0, keepdims=True)
    con_ref[0] = lse - picked

    maskf = (labels > 0).astype(jnp.float32)

    p = ploc_ref[0]
    g = gloc_ref[0]
    d = dboxes_ref[0]
    gxy = SCALE_XY * (g[:2, :] - d[:2, :]) / d[2:, :]
    gwh = SCALE_WH * jnp.log(g[2:, :] / d[2:, :])
    dxy = p[:2, :] - gxy
    dwh = p[2:, :] - gwh
    diff = jnp.concatenate([dxy, dwh], axis=0)
    ad = jnp.abs(diff)
    sl1 = jnp.sum(jnp.where(ad < 1.0, 0.5 * diff * diff, ad - 0.5), axis=0,
                  keepdims=True)
    sl1_ref[0] = jnp.sum(maskf * sl1, axis=1, keepdims=True)


def _mine_kernel(con_ref, glabel_ref, sl1_ref, out_ref):
    con = con_ref[...]
    labels = glabel_ref[...]
    mask = labels > 0
    maskf = mask.astype(jnp.float32)
    pos = jnp.sum(maskf, axis=1, keepdims=True).astype(jnp.int32)
    K = jnp.minimum(3 * pos, jnp.int32(L))

    con_neg = jnp.where(mask, 0.0, con)
    keys = _sortable_key(con_neg)

    cnt_nonneg = jnp.sum((keys >= 0).astype(jnp.int32), axis=1, keepdims=True)
    big = cnt_nonneg >= K
    lo0 = jnp.where(big, jnp.int32(0), jnp.int32(INT32_MIN))
    hi0 = jnp.where(big, jnp.int32(INT32_MAX), jnp.int32(-1))

    def val_body(_, carry):
        lo, hi = carry
        span = hi - lo
        mid = lo + (span >> 1) + (span & 1)
        cnt = jnp.sum((keys >= mid).astype(jnp.int32), axis=1, keepdims=True)
        ok = cnt >= K
        return jnp.where(ok, mid, lo), jnp.where(ok, hi, mid - 1)

    t, _ = jax.lax.fori_loop(0, 31, val_body, (lo0, hi0))

    gt = keys > t
    count_gt = jnp.sum(gt.astype(jnp.int32), axis=1, keepdims=True)
    need = K - count_gt

    tied = keys == t
    tied_i32 = tied.astype(jnp.int32)
    idx = jax.lax.broadcasted_iota(jnp.int32, (N, L), 1)

    def idx_body(_, carry):
        lo, hi = carry
        mid = (lo + hi) >> 1
        cnt = jnp.sum(jnp.where(idx < mid, tied_i32, 0), axis=1, keepdims=True)
        ok = cnt >= need
        return jnp.where(ok, lo, mid + 1), jnp.where(ok, mid, hi)

    msel, _ = jax.lax.fori_loop(
        0, 14, idx_body,
        (jnp.zeros((N, 1), jnp.int32), jnp.full((N, 1), L, jnp.int32)))

    neg_sel = jnp.logical_or(gt, jnp.logical_and(tied, idx < msel))
    closs = jnp.sum(con * (maskf + neg_sel.astype(jnp.float32)), axis=1,
                    keepdims=True)

    total = sl1_ref[...] + closs
    posf = pos.astype(jnp.float32)
    num_mask = (pos > 0).astype(jnp.float32)
    contrib = total * num_mask / jnp.maximum(posf, 1e-6)
    out_ref[...] = jnp.sum(contrib, axis=0, keepdims=True) * (1.0 / N)


@jax.jit
def kernel(ploc, plabel, gloc, glabel, dboxes):
    ploc = ploc.astype(jnp.float32)
    plabel = plabel.astype(jnp.float32)
    glabel3 = glabel.reshape(N, 1, L).astype(jnp.int32)

    con3, sl13 = pl.pallas_call(
        _dense_kernel,
        grid=(N,),
        in_specs=[
            pl.BlockSpec((1, C, L), lambda i: (i, 0, 0)),
            pl.BlockSpec((1, 1, L), lambda i: (i, 0, 0)),
            pl.BlockSpec((1, 4, L), lambda i: (i, 0, 0)),
            pl.BlockSpec((1, 4, L), lambda i: (i, 0, 0)),
            pl.BlockSpec((1, 4, L), lambda i: (0, 0, 0)),
        ],
        out_specs=[
            pl.BlockSpec((1, 1, L), lambda i: (i, 0, 0)),
            pl.BlockSpec((1, 1, 1), lambda i: (i, 0, 0)),
        ],
        out_shape=[
            jax.ShapeDtypeStruct((N, 1, L), jnp.float32),
            jax.ShapeDtypeStruct((N, 1, 1), jnp.float32),
        ],
    )(plabel, glabel3, ploc, gloc, dboxes)

    out = pl.pallas_call(
        _mine_kernel,
        grid=(1,),
        in_specs=[
            pl.BlockSpec((N, L), lambda i: (0, 0)),
            pl.BlockSpec((N, L), lambda i: (0, 0)),
            pl.BlockSpec((N, 1), lambda i: (0, 0)),
        ],
        out_specs=pl.BlockSpec((1, 1), lambda i: (0, 0)),
        out_shape=jax.ShapeDtypeStruct((1, 1), jnp.float32),
    )(con3.reshape(N, L), glabel.astype(jnp.int32), sl13.reshape(N, 1))
    return out[0, 0]


# speedup vs baseline: 4.9881x; 1.0360x over previous
import jax
import jax.numpy as jnp
from jax.experimental import pallas as pl

N, C, L = 64, 81, 8732
SCALE_XY = 10.0
SCALE_WH = 5.0
INT32_MIN = -2147483648
INT32_MAX = 2147483647


def _sortable_key(f):
    b = jax.lax.bitcast_convert_type(f, jnp.int32)
    return jnp.where(b >= 0, b, jnp.int32(INT32_MIN) - b)


R = 4


def _dense_kernel(plabel_ref, glabel_ref, ploc_ref, gloc_ref, dboxes_ref,
                  con_ref, sl1_ref):
    x = plabel_ref[...]
    labels = glabel_ref[...]

    m = jnp.max(x, axis=1, keepdims=True)
    s = jnp.sum(jnp.exp(x - m), axis=1, keepdims=True)
    lse = jnp.log(s) + m

    cls = jax.lax.broadcasted_iota(jnp.int32, (R, C, L), 1)
    onehot = (cls == labels).astype(jnp.float32)
    picked = jnp.sum(onehot * x, axis=1, keepdims=True)
    con_ref[...] = lse - picked

    maskf = (labels > 0).astype(jnp.float32)

    p = ploc_ref[...]
    g = gloc_ref[...]
    d = dboxes_ref[...]
    gxy = SCALE_XY * (g[:, :2, :] - d[:, :2, :]) / d[:, 2:, :]
    gwh = SCALE_WH * jnp.log(g[:, 2:, :] / d[:, 2:, :])
    dxy = p[:, :2, :] - gxy
    dwh = p[:, 2:, :] - gwh
    diff = jnp.concatenate([dxy, dwh], axis=1)
    ad = jnp.abs(diff)
    sl1 = jnp.sum(jnp.where(ad < 1.0, 0.5 * diff * diff, ad - 0.5), axis=1,
                  keepdims=True)
    sl1_ref[...] = jnp.sum(maskf * sl1, axis=2, keepdims=True)


def _mine_kernel(con_ref, glabel_ref, sl1_ref, out_ref):
    con = con_ref[...]
    labels = glabel_ref[...]
    mask = labels > 0
    maskf = mask.astype(jnp.float32)
    pos = jnp.sum(maskf, axis=1, keepdims=True).astype(jnp.int32)
    K = jnp.minimum(3 * pos, jnp.int32(L))

    con_neg = jnp.where(mask, 0.0, con)
    keys = _sortable_key(con_neg)

    cnt_nonneg = jnp.sum((keys >= 0).astype(jnp.int32), axis=1, keepdims=True)
    big = cnt_nonneg >= K
    lo0 = jnp.where(big, jnp.int32(0), jnp.int32(INT32_MIN))
    hi0 = jnp.where(big, jnp.int32(INT32_MAX), jnp.int32(-1))

    def val_body(_, carry):
        lo, hi = carry
        span = hi - lo
        mid = lo + (span >> 1) + (span & 1)
        cnt = jnp.sum((keys >= mid).astype(jnp.int32), axis=1, keepdims=True)
        ok = cnt >= K
        return jnp.where(ok, mid, lo), jnp.where(ok, hi, mid - 1)

    t, _ = jax.lax.fori_loop(0, 31, val_body, (lo0, hi0))

    gt = keys > t
    count_gt = jnp.sum(gt.astype(jnp.int32), axis=1, keepdims=True)
    need = K - count_gt

    tied = keys == t
    tied_i32 = tied.astype(jnp.int32)
    idx = jax.lax.broadcasted_iota(jnp.int32, (N, L), 1)

    def idx_body(_, carry):
        lo, hi = carry
        mid = (lo + hi) >> 1
        cnt = jnp.sum(jnp.where(idx < mid, tied_i32, 0), axis=1, keepdims=True)
        ok = cnt >= need
        return jnp.where(ok, lo, mid + 1), jnp.where(ok, mid, hi)

    msel, _ = jax.lax.fori_loop(
        0, 14, idx_body,
        (jnp.zeros((N, 1), jnp.int32), jnp.full((N, 1), L, jnp.int32)))

    neg_sel = jnp.logical_or(gt, jnp.logical_and(tied, idx < msel))
    closs = jnp.sum(con * (maskf + neg_sel.astype(jnp.float32)), axis=1,
                    keepdims=True)

    total = sl1_ref[...] + closs
    posf = pos.astype(jnp.float32)
    num_mask = (pos > 0).astype(jnp.float32)
    contrib = total * num_mask / jnp.maximum(posf, 1e-6)
    out_ref[...] = jnp.sum(contrib, axis=0, keepdims=True) * (1.0 / N)


@jax.jit
def kernel(ploc, plabel, gloc, glabel, dboxes):
    ploc = ploc.astype(jnp.float32)
    plabel = plabel.astype(jnp.float32)
    glabel3 = glabel.reshape(N, 1, L).astype(jnp.int32)

    con3, sl13 = pl.pallas_call(
        _dense_kernel,
        grid=(N // R,),
        in_specs=[
            pl.BlockSpec((R, C, L), lambda i: (i, 0, 0)),
            pl.BlockSpec((R, 1, L), lambda i: (i, 0, 0)),
            pl.BlockSpec((R, 4, L), lambda i: (i, 0, 0)),
            pl.BlockSpec((R, 4, L), lambda i: (i, 0, 0)),
            pl.BlockSpec((1, 4, L), lambda i: (0, 0, 0)),
        ],
        out_specs=[
            pl.BlockSpec((R, 1, L), lambda i: (i, 0, 0)),
            pl.BlockSpec((R, 1, 1), lambda i: (i, 0, 0)),
        ],
        out_shape=[
            jax.ShapeDtypeStruct((N, 1, L), jnp.float32),
            jax.ShapeDtypeStruct((N, 1, 1), jnp.float32),
        ],
    )(plabel, glabel3, ploc, gloc, dboxes)

    out = pl.pallas_call(
        _mine_kernel,
        grid=(1,),
        in_specs=[
            pl.BlockSpec((N, L), lambda i: (0, 0)),
            pl.BlockSpec((N, L), lambda i: (0, 0)),
            pl.BlockSpec((N, 1), lambda i: (0, 0)),
        ],
        out_specs=pl.BlockSpec((1, 1), lambda i: (0, 0)),
        out_shape=jax.ShapeDtypeStruct((1, 1), jnp.float32),
    )(con3.reshape(N, L), glabel.astype(jnp.int32), sl13.reshape(N, 1))
    return out[0, 0]
